# baseline (speedup 1.0000x reference)
import functools

import jax
import jax.numpy as jnp
from jax import lax
from jax.experimental import pallas as pl
from jax.experimental.pallas import tpu as pltpu
from jax.experimental.pallas import tpu_sc as plsc

N = 10000
D = 128
E = 320000
NSETS = 4

NC = 2
NS = 16
K = 128
KD = 80
EPT = 20480
NCH = EPT // K
ACC_N = 10240
ZCH = ACC_N // NS


def _artanh(x):
    x = jnp.clip(x, -1.0 + 1e-7, 1.0 - 1e-7)
    return 0.5 * jnp.log((1.0 + x) / (1.0 - x))


def _nrm(x):
    return jnp.clip(jnp.sqrt(jnp.sum(x * x, -1, keepdims=True)), 1e-15)


def _proj(x):
    maxnorm = 1.0 - 1e-3
    n = _nrm(x)
    return jnp.where(n > maxnorm, x / n * maxnorm, x)


def _expmap0(u):
    un = _nrm(u)
    return jnp.tanh(un) * u / un


def _logmap0(p):
    pn = _nrm(p)
    return _artanh(pn) * p / pn


def _mobius_add(x, y):
    x2 = jnp.sum(x * x, -1, keepdims=True)
    y2 = jnp.sum(y * y, -1, keepdims=True)
    xy = jnp.sum(x * y, -1, keepdims=True)
    num = (1.0 + 2.0 * xy + y2) * x + (1.0 - x2) * y
    den = 1.0 + 2.0 * xy + x2 * y2
    return num / jnp.clip(den, 1e-15)


def _hyp_linear(h, wt, bvec):
    xn = _nrm(h)
    mx = jnp.dot(h, wt, preferred_element_type=jnp.float32)
    mxn = _nrm(mx)
    mv = _proj(jnp.tanh(mxn / xn * _artanh(xn)) * mx / mxn)
    hb = _proj(_expmap0(bvec[None, :]))
    res = _proj(_mobius_add(mv, hb))
    return _logmap0(res)


def _post_agg(agg_raw, w):
    agg = agg_raw * w[:, None]
    h2 = _proj(_expmap0(agg))
    xt2 = jax.nn.relu(_logmap0(h2))
    return _proj(_expmap0(xt2))


R = 1024
GRID = ACC_N // R


def _tc1_body(x_ref, wt_ref, b_ref, out_ref):
    h = _proj(_expmap0(x_ref[...]))
    xn = _nrm(h)
    mx_all = jnp.dot(h, wt_ref[...], preferred_element_type=jnp.float32)
    for s in range(NSETS):
        mx = mx_all[:, s * D:(s + 1) * D]
        mxn = _nrm(mx)
        mv = _proj(jnp.tanh(mxn / xn * _artanh(xn)) * mx / mxn)
        hb = _proj(_expmap0(b_ref[s][None, :]))
        res = _proj(_mobius_add(mv, hb))
        out_ref[s] = _logmap0(res)


def _tc2_body(agg_ref, deg_ref, wt_ref, b_ref, out_ref):
    w = 1.0 / jnp.clip(deg_ref[...], 1.0)
    for s in range(NSETS):
        h = _post_agg(agg_ref[s], w[s])
        out_ref[s] = _hyp_linear(h, wt_ref[s], b_ref[s])


def _tc3_body(agg_ref, deg_ref, out_ref):
    w = 1.0 / jnp.clip(deg_ref[...], 1.0)
    x1 = [_post_agg(agg_ref[s], w[s]) for s in range(NSETS)]
    wgt = [_proj(_expmap0(0.125 * _logmap0(xi))) for xi in x1]
    target = _mobius_add(_mobius_add(_mobius_add(wgt[0], wgt[1]), wgt[2]), wgt[3])
    tt = [_logmap0(v) for v in (x1 + [target])]
    mean = (tt[0] + tt[1] + tt[2] + tt[3] + tt[4]) * 0.2
    out_ref[...] = _proj(_expmap0(mean))


def _tc1(x, wt_all, b0):
    return pl.pallas_call(
        _tc1_body,
        grid=(GRID,),
        in_specs=[
            pl.BlockSpec((R, D), lambda i: (i, 0)),
            pl.BlockSpec((D, NSETS * D), lambda i: (0, 0)),
            pl.BlockSpec((NSETS, D), lambda i: (0, 0)),
        ],
        out_specs=pl.BlockSpec((NSETS, R, D), lambda i: (0, i, 0)),
        out_shape=jax.ShapeDtypeStruct((NSETS, ACC_N, D), jnp.float32),
    )(x, wt_all, b0)


def _tc2(agg, deg, wt2, b1):
    return pl.pallas_call(
        _tc2_body,
        grid=(GRID,),
        in_specs=[
            pl.BlockSpec((NSETS, R, D), lambda i: (0, i, 0)),
            pl.BlockSpec((NSETS, R), lambda i: (0, i)),
            pl.BlockSpec((NSETS, D, D), lambda i: (0, 0, 0)),
            pl.BlockSpec((NSETS, D), lambda i: (0, 0)),
        ],
        out_specs=pl.BlockSpec((NSETS, R, D), lambda i: (0, i, 0)),
        out_shape=jax.ShapeDtypeStruct((NSETS, ACC_N, D), jnp.float32),
    )(agg, deg, wt2, b1)


def _tc3(agg, deg):
    return pl.pallas_call(
        _tc3_body,
        grid=(GRID,),
        in_specs=[
            pl.BlockSpec((NSETS, R, D), lambda i: (0, i, 0)),
            pl.BlockSpec((NSETS, R), lambda i: (0, i)),
        ],
        out_specs=pl.BlockSpec((R, D), lambda i: (i, 0)),
        out_shape=jax.ShapeDtypeStruct((N, D), jnp.float32),
    )(agg, deg)


GC = 32
NGRP = NCH // GC
GCD = 50


def _sc_agg_body(xt_hbm, src_hbm, dst_hbm, agg_hbm,
                 acc, sbuf, dbuf, r0, r1, g0, g1):
    cid = lax.axis_index("c")
    tid = lax.axis_index("s")

    for si in range(2):
        s = cid * 2 + si

        def _zero_rows(i, _):
            r0[i // 8, pl.ds((i % 8) * 16, 16)] = jnp.zeros((16,), jnp.float32)
            return 0
        lax.fori_loop(0, (K * D) // 16, _zero_rows, 0)
        for q in range(ZCH // K):
            pltpu.sync_copy(r0, acc.at[pl.ds(tid * ZCH + q * K, K), :])
        rem = ZCH - (ZCH // K) * K
        if rem:
            pltpu.sync_copy(r0.at[pl.ds(0, rem), :],
                            acc.at[pl.ds(tid * ZCH + (ZCH // K) * K, rem), :])
        plsc.subcore_barrier()

        for gl in range(NGRP):
            pltpu.sync_copy(src_hbm.at[s, tid, gl], sbuf)
            pltpu.sync_copy(dst_hbm.at[s, tid, gl], dbuf)
            pltpu.async_copy(xt_hbm.at[sbuf.at[0]], r0, g0)

            def _chunk_pair(w, _):
                a = 2 * w
                b = a + 1
                pltpu.make_async_copy(xt_hbm.at[sbuf.at[a]], r0, g0).wait()
                pltpu.async_copy(xt_hbm.at[sbuf.at[b]], r1, g1)
                pltpu.sync_copy(r0, acc.at[dbuf.at[a]], add=True)
                pltpu.make_async_copy(xt_hbm.at[sbuf.at[b]], r1, g1).wait()

                @pl.when(w < GC // 2 - 1)
                def _():
                    pltpu.async_copy(xt_hbm.at[sbuf.at[a + 2]], r0, g0)

                pltpu.sync_copy(r1, acc.at[dbuf.at[b]], add=True)
                return 0

            lax.fori_loop(0, GC // 2, _chunk_pair, 0)

        plsc.subcore_barrier()
        pltpu.sync_copy(acc.at[pl.ds(tid * ZCH, ZCH), :],
                        agg_hbm.at[s, pl.ds(tid * ZCH, ZCH), :])
        plsc.subcore_barrier()


def _sc_deg_body(dst_hbm, deg3_hbm, stage, ldeg, rbuf, obuf, dbuf):
    cid = lax.axis_index("c")
    tid = lax.axis_index("s")

    ones16 = jnp.ones((16,), jnp.float32)
    zeros16 = jnp.zeros((16,), jnp.float32)

    for si in range(2):
        s = cid * 2 + si

        def _zero(i, _):
            ldeg[pl.ds(i * 16, 16)] = zeros16
            return 0
        lax.fori_loop(0, ACC_N // 16, _zero, 0)

        for gl in range(NGRP):
            pltpu.sync_copy(dst_hbm.at[s, tid, gl], dbuf)

            def _count(f, _):
                iv = dbuf[f // 5, pl.ds((f % 5) * 16, 16)]
                plsc.addupdate_scatter(ldeg, [iv], ones16)
                return 0

            lax.fori_loop(0, (GCD * KD) // 16, _count, 0)

        pltpu.sync_copy(ldeg, stage.at[tid, 0])
        plsc.subcore_barrier()

        @pl.when(tid < ACC_N // 1024)
        def _():
            pltpu.sync_copy(stage.at[:, 0, pl.ds(tid * 1024, 1024)], rbuf)

            def _reduce(f, _):
                c = f * 16
                tot = rbuf[0, pl.ds(c, 16)]
                for t in range(1, NS):
                    tot = tot + rbuf[t, pl.ds(c, 16)]
                obuf[f // 8, pl.ds((f % 8) * 16, 16)] = tot
                return 0

            lax.fori_loop(0, 1024 // 16, _reduce, 0)
            pltpu.sync_copy(obuf, deg3_hbm.at[s, pl.ds(tid * 8, 8), :])

        plsc.subcore_barrier()


def _sc_mesh():
    return plsc.VectorSubcoreMesh(core_axis_name="c", subcore_axis_name="s",
                                  num_cores=NC, num_subcores=NS)


def _sc_agg(xt2d, srcoff, dstr):
    fn = pl.kernel(
        _sc_agg_body,
        out_type=(jax.ShapeDtypeStruct((NSETS, ACC_N, D), jnp.float32),),
        mesh=_sc_mesh(),
        scratch_types=[
            pltpu.VMEM_SHARED((ACC_N, D), jnp.float32),
            pltpu.VMEM((GC, K), jnp.int32),
            pltpu.VMEM((GC, K), jnp.int32),
            pltpu.VMEM((K, D), jnp.float32),
            pltpu.VMEM((K, D), jnp.float32),
            pltpu.SemaphoreType.DMA,
            pltpu.SemaphoreType.DMA,
        ],
    )
    return fn(xt2d, srcoff, dstr)[0]


def _sc_deg(dstr):
    fn = pl.kernel(
        _sc_deg_body,
        out_type=(jax.ShapeDtypeStruct((NSETS, ACC_N // 128, 128), jnp.float32),),
        mesh=_sc_mesh(),
        compiler_params=pltpu.CompilerParams(needs_layout_passes=False),
        scratch_types=[
            pltpu.VMEM_SHARED((NS, 1, ACC_N), jnp.float32),
            pltpu.VMEM((ACC_N,), jnp.float32),
            pltpu.VMEM((NS, 1024), jnp.float32),
            pltpu.VMEM((8, 128), jnp.float32),
            pltpu.VMEM((GCD, KD), jnp.int32),
        ],
    )
    return fn(dstr)[0]


def _jnp_agg(xt2d, srcoff, dstr):
    xt = xt2d.reshape(NSETS, ACC_N, D)
    srcf = srcoff.reshape(NSETS, E) - (jnp.arange(NSETS, dtype=jnp.int32) * ACC_N)[:, None]
    dstf = dstr.reshape(NSETS, E)
    return jnp.stack([
        jax.ops.segment_sum(jnp.take(xt[s], srcf[s], axis=0), dstf[s], num_segments=ACC_N)
        for s in range(NSETS)])


def _jnp_deg(dstr):
    dstf = dstr.reshape(NSETS, E)
    return jnp.stack([
        jax.ops.segment_sum(jnp.ones((E,), jnp.float32), dstf[s],
                            num_segments=ACC_N)
        for s in range(NSETS)]).reshape(NSETS, ACC_N // 128, 128)


DBG_SC_DEG = True
DBG_SC_AGG = True


def kernel(x, W, b, ei_d_i, ei_d_o, ei_n_i, ei_n_o):
    ei = jnp.stack([ei_d_i, ei_d_o, ei_n_i, ei_n_o])
    npad = EPT - E // NS
    srcp = jnp.pad(ei[:, 0, :].reshape(NSETS, NS, E // NS),
                   ((0, 0), (0, 0), (0, npad)))
    srcoff = srcp + (jnp.arange(NSETS, dtype=jnp.int32) * ACC_N)[:, None, None]
    srcoff = srcoff.reshape(NSETS, NS, NGRP, GC, K)
    padd = (N + jnp.arange(npad, dtype=jnp.int32) % (ACC_N - N))[None, None, :]
    dstr = jnp.concatenate(
        [ei[:, 1, :].reshape(NSETS, NS, E // NS),
         jnp.broadcast_to(padd, (NSETS, NS, npad))], axis=2)
    dstr = dstr.reshape(NSETS, NS, NGRP, GC, K)
    dstr_deg = ei[:, 1, :].reshape(NSETS, NS, NGRP, GCD, KD)

    wt1 = jnp.concatenate([W[s, 0].T for s in range(NSETS)], axis=1)
    wt2 = W[:, 1].transpose(0, 2, 1)

    f_deg = _sc_deg if DBG_SC_DEG else _jnp_deg
    f_agg = _sc_agg if DBG_SC_AGG else _jnp_agg
    deg = f_deg(dstr_deg).reshape(NSETS, ACC_N)
    xt0 = _tc1(x, wt1, b[:, 0])
    agg1 = f_agg(xt0.reshape(NSETS * ACC_N, D), srcoff, dstr)
    xt1 = _tc2(agg1, deg, wt2, b[:, 1])
    agg2 = f_agg(xt1.reshape(NSETS * ACC_N, D), srcoff, dstr)
    return _tc3(agg2, deg)


# speedup vs baseline: 1.8199x; 1.8199x over previous
import functools

import jax
import jax.numpy as jnp
from jax import lax
from jax.experimental import pallas as pl
from jax.experimental.pallas import tpu as pltpu
from jax.experimental.pallas import tpu_sc as plsc

N = 10000
D = 128
E = 320000
NSETS = 4

NC = 2
NS = 16
K = 100
KD = 80
NCH = E // (NS * K)
ACC_N = 10240
ZCH = ACC_N // NS


def _artanh(x):
    x = jnp.clip(x, -1.0 + 1e-7, 1.0 - 1e-7)
    return 0.5 * jnp.log((1.0 + x) / (1.0 - x))


def _nrm(x):
    return jnp.clip(jnp.sqrt(jnp.sum(x * x, -1, keepdims=True)), 1e-15)


def _proj(x):
    maxnorm = 1.0 - 1e-3
    n = _nrm(x)
    return jnp.where(n > maxnorm, x / n * maxnorm, x)


def _expmap0(u):
    un = _nrm(u)
    return jnp.tanh(un) * u / un


def _logmap0(p):
    pn = _nrm(p)
    return _artanh(pn) * p / pn


def _mobius_add(x, y):
    x2 = jnp.sum(x * x, -1, keepdims=True)
    y2 = jnp.sum(y * y, -1, keepdims=True)
    xy = jnp.sum(x * y, -1, keepdims=True)
    num = (1.0 + 2.0 * xy + y2) * x + (1.0 - x2) * y
    den = 1.0 + 2.0 * xy + x2 * y2
    return num / jnp.clip(den, 1e-15)


def _hyp_linear(h, wt, bvec):
    xn = _nrm(h)
    mx = jnp.dot(h, wt, preferred_element_type=jnp.float32)
    mxn = _nrm(mx)
    mv = _proj(jnp.tanh(mxn / xn * _artanh(xn)) * mx / mxn)
    hb = _proj(_expmap0(bvec[None, :]))
    res = _proj(_mobius_add(mv, hb))
    return _logmap0(res)


def _post_agg(agg_raw, w):
    agg = agg_raw * w[:, None]
    h2 = _proj(_expmap0(agg))
    xt2 = jax.nn.relu(_logmap0(h2))
    return _proj(_expmap0(xt2))


R = 1024
GRID = ACC_N // R


def _tc1_body(x_ref, wt_ref, b_ref, outa_ref, outb_ref):
    h = _proj(_expmap0(x_ref[...]))
    xn = _nrm(h)
    mx_all = jnp.dot(h, wt_ref[...], preferred_element_type=jnp.float32)
    for s in range(NSETS):
        mx = mx_all[:, s * D:(s + 1) * D]
        mxn = _nrm(mx)
        mv = _proj(jnp.tanh(mxn / xn * _artanh(xn)) * mx / mxn)
        hb = _proj(_expmap0(b_ref[s][None, :]))
        res = _proj(_mobius_add(mv, hb))
        if s < 2:
            outa_ref[s] = _logmap0(res)
        else:
            outb_ref[s - 2] = _logmap0(res)


def _tc2_body(agg_ref, deg_ref, wt_ref, b_ref, out_ref):
    w = 1.0 / jnp.clip(deg_ref[...], 1.0)
    for s in range(2):
        h = _post_agg(agg_ref[s], w[s])
        out_ref[s] = _hyp_linear(h, wt_ref[s], b_ref[s])


def _tc3_body(agga_ref, aggb_ref, deg_ref, out_ref):
    w = 1.0 / jnp.clip(deg_ref[...], 1.0)
    x1 = ([_post_agg(agga_ref[s], w[s]) for s in range(2)]
          + [_post_agg(aggb_ref[s], w[s + 2]) for s in range(2)])
    wgt = [_proj(_expmap0(0.125 * _logmap0(xi))) for xi in x1]
    target = _mobius_add(_mobius_add(_mobius_add(wgt[0], wgt[1]), wgt[2]), wgt[3])
    tt = [_logmap0(v) for v in (x1 + [target])]
    mean = (tt[0] + tt[1] + tt[2] + tt[3] + tt[4]) * 0.2
    out_ref[...] = _proj(_expmap0(mean))


def _tc1(x, wt_all, b0):
    return pl.pallas_call(
        _tc1_body,
        grid=(GRID,),
        in_specs=[
            pl.BlockSpec((R, D), lambda i: (i, 0)),
            pl.BlockSpec((D, NSETS * D), lambda i: (0, 0)),
            pl.BlockSpec((NSETS, D), lambda i: (0, 0)),
        ],
        out_specs=[pl.BlockSpec((2, R, D), lambda i: (0, i, 0)),
                   pl.BlockSpec((2, R, D), lambda i: (0, i, 0))],
        out_shape=[jax.ShapeDtypeStruct((2, ACC_N, D), jnp.float32),
                   jax.ShapeDtypeStruct((2, ACC_N, D), jnp.float32)],
    )(x, wt_all, b0)


def _tc2(agg, deg, wt2, b1):
    return pl.pallas_call(
        _tc2_body,
        grid=(GRID,),
        in_specs=[
            pl.BlockSpec((2, R, D), lambda i: (0, i, 0)),
            pl.BlockSpec((2, R), lambda i: (0, i)),
            pl.BlockSpec((2, D, D), lambda i: (0, 0, 0)),
            pl.BlockSpec((2, D), lambda i: (0, 0)),
        ],
        out_specs=pl.BlockSpec((2, R, D), lambda i: (0, i, 0)),
        out_shape=jax.ShapeDtypeStruct((2, ACC_N, D), jnp.float32),
    )(agg, deg, wt2, b1)


def _tc3(agga, aggb, deg):
    return pl.pallas_call(
        _tc3_body,
        grid=(GRID,),
        in_specs=[
            pl.BlockSpec((2, R, D), lambda i: (0, i, 0)),
            pl.BlockSpec((2, R, D), lambda i: (0, i, 0)),
            pl.BlockSpec((NSETS, R), lambda i: (0, i)),
        ],
        out_specs=pl.BlockSpec((R, D), lambda i: (i, 0)),
        out_shape=jax.ShapeDtypeStruct((N, D), jnp.float32),
    )(agga, aggb, deg)


GC = 40
NGRP = NCH // GC
GCD = 50


def _sc_agg_body(xt_hbm, src_hbm, dst_hbm, agg_hbm,
                 acc, sbuf, dbuf, r0, r1, g0, g1):
    cid = lax.axis_index("c")
    tid = lax.axis_index("s")

    for si in range(1):
        s = cid + si

        def _zero_rows(i, _):
            r0[i // 8, pl.ds((i % 8) * 16, 16)] = jnp.zeros((16,), jnp.float32)
            return 0
        lax.fori_loop(0, (K * D) // 16, _zero_rows, 0)
        for q in range(ZCH // K):
            pltpu.sync_copy(r0, acc.at[pl.ds(tid * ZCH + q * K, K), :])
        rem = ZCH - (ZCH // K) * K
        if rem:
            pltpu.sync_copy(r0.at[pl.ds(0, rem), :],
                            acc.at[pl.ds(tid * ZCH + (ZCH // K) * K, rem), :])
        plsc.subcore_barrier()

        for gl in range(NGRP):
            pltpu.sync_copy(src_hbm.at[s, tid, gl], sbuf)
            pltpu.sync_copy(dst_hbm.at[s, tid, gl], dbuf)
            pltpu.async_copy(xt_hbm.at[sbuf.at[0]], r0, g0)

            def _chunk_pair(w, _):
                a = 2 * w
                b = a + 1
                pltpu.make_async_copy(xt_hbm.at[sbuf.at[a]], r0, g0).wait()
                pltpu.async_copy(xt_hbm.at[sbuf.at[b]], r1, g1)
                pltpu.sync_copy(r0, acc.at[dbuf.at[a]], add=True)
                pltpu.make_async_copy(xt_hbm.at[sbuf.at[b]], r1, g1).wait()

                @pl.when(w < GC // 2 - 1)
                def _():
                    pltpu.async_copy(xt_hbm.at[sbuf.at[a + 2]], r0, g0)

                pltpu.sync_copy(r1, acc.at[dbuf.at[b]], add=True)
                return 0

            lax.fori_loop(0, GC // 2, _chunk_pair, 0)

        plsc.subcore_barrier()
        pltpu.sync_copy(acc.at[pl.ds(tid * ZCH, ZCH), :],
                        agg_hbm.at[s, pl.ds(tid * ZCH, ZCH), :])
        plsc.subcore_barrier()


def _sc_deg_body(dst_hbm, deg3_hbm, stage, ldeg, rbuf, obuf, dbuf):
    cid = lax.axis_index("c")
    tid = lax.axis_index("s")

    ones16 = jnp.ones((16,), jnp.float32)
    zeros16 = jnp.zeros((16,), jnp.float32)

    for si in range(2):
        s = cid * 2 + si

        def _zero(i, _):
            ldeg[pl.ds(i * 16, 16)] = zeros16
            return 0
        lax.fori_loop(0, ACC_N // 16, _zero, 0)

        for gl in range(NGRP):
            pltpu.sync_copy(dst_hbm.at[s, tid, gl], dbuf)

            def _count(f, _):
                iv = dbuf[f // 5, pl.ds((f % 5) * 16, 16)]
                plsc.addupdate_scatter(ldeg, [iv], ones16)
                return 0

            lax.fori_loop(0, (GCD * KD) // 16, _count, 0)

        pltpu.sync_copy(ldeg, stage.at[tid, 0])
        plsc.subcore_barrier()

        @pl.when(tid < ACC_N // 1024)
        def _():
            pltpu.sync_copy(stage.at[:, 0, pl.ds(tid * 1024, 1024)], rbuf)

            def _reduce(f, _):
                c = f * 16
                tot = rbuf[0, pl.ds(c, 16)]
                for t in range(1, NS):
                    tot = tot + rbuf[t, pl.ds(c, 16)]
                obuf[f // 8, pl.ds((f % 8) * 16, 16)] = tot
                return 0

            lax.fori_loop(0, 1024 // 16, _reduce, 0)
            pltpu.sync_copy(obuf, deg3_hbm.at[s, pl.ds(tid * 8, 8), :])

        plsc.subcore_barrier()


def _sc_mesh():
    return plsc.VectorSubcoreMesh(core_axis_name="c", subcore_axis_name="s",
                                  num_cores=NC, num_subcores=NS)


def _sc_agg(xt2d, srcoff, dstr):
    fn = pl.kernel(
        _sc_agg_body,
        out_type=(jax.ShapeDtypeStruct((2, ACC_N, D), jnp.float32),),
        mesh=_sc_mesh(),
        scratch_types=[
            pltpu.VMEM_SHARED((ACC_N, D), jnp.float32),
            pltpu.VMEM((GC, K), jnp.int32),
            pltpu.VMEM((GC, K), jnp.int32),
            pltpu.VMEM((K, D), jnp.float32),
            pltpu.VMEM((K, D), jnp.float32),
            pltpu.SemaphoreType.DMA,
            pltpu.SemaphoreType.DMA,
        ],
    )
    return fn(xt2d, srcoff, dstr)[0]


def _sc_deg(dstr):
    fn = pl.kernel(
        _sc_deg_body,
        out_type=(jax.ShapeDtypeStruct((NSETS, ACC_N // 128, 128), jnp.float32),),
        mesh=_sc_mesh(),
        compiler_params=pltpu.CompilerParams(needs_layout_passes=False),
        scratch_types=[
            pltpu.VMEM_SHARED((NS, 1, ACC_N), jnp.float32),
            pltpu.VMEM((ACC_N,), jnp.float32),
            pltpu.VMEM((NS, 1024), jnp.float32),
            pltpu.VMEM((8, 128), jnp.float32),
            pltpu.VMEM((GCD, KD), jnp.int32),
        ],
    )
    return fn(dstr)[0]


def kernel(x, W, b, ei_d_i, ei_d_o, ei_n_i, ei_n_o):
    ei = jnp.stack([ei_d_i, ei_d_o, ei_n_i, ei_n_o])
    srcoff = (ei[:, 0, :]
              + ((jnp.arange(NSETS, dtype=jnp.int32) % 2) * ACC_N)[:, None])
    srcoff = srcoff.reshape(NSETS, NS, NGRP, GC, K)
    dstr = ei[:, 1, :].reshape(NSETS, NS, NGRP, GC, K)
    dstr_deg = ei[:, 1, :].reshape(NSETS, NS, NGRP, GCD, KD)

    wt1 = jnp.concatenate([W[s, 0].T for s in range(NSETS)], axis=1)
    wt2 = W[:, 1].transpose(0, 2, 1)

    deg = _sc_deg(dstr_deg).reshape(NSETS, ACC_N)
    xt0a, xt0b = _tc1(x, wt1, b[:, 0])
    agg1a = _sc_agg(xt0a.reshape(2 * ACC_N, D), srcoff[0:2], dstr[0:2])
    agg1b = _sc_agg(xt0b.reshape(2 * ACC_N, D), srcoff[2:4], dstr[2:4])
    xt1a = _tc2(agg1a, deg[0:2], wt2[0:2], b[0:2, 1])
    xt1b = _tc2(agg1b, deg[2:4], wt2[2:4], b[2:4, 1])
    agg2a = _sc_agg(xt1a.reshape(2 * ACC_N, D), srcoff[0:2], dstr[0:2])
    agg2b = _sc_agg(xt1b.reshape(2 * ACC_N, D), srcoff[2:4], dstr[2:4])
    return _tc3(agg2a, agg2b, deg)


# speedup vs baseline: 1.8884x; 1.0376x over previous
import functools

import jax
import jax.numpy as jnp
from jax import lax
from jax.experimental import pallas as pl
from jax.experimental.pallas import tpu as pltpu
from jax.experimental.pallas import tpu_sc as plsc

N = 10000
D = 128
E = 320000
NSETS = 4

NC = 2
NS = 16
K = 100
KD = 80
NCH = E // (NS * K)
ACC_N = 10240
ZCH = ACC_N // NS


def _artanh(x):
    x = jnp.clip(x, -1.0 + 1e-7, 1.0 - 1e-7)
    return 0.5 * jnp.log((1.0 + x) / (1.0 - x))


def _nrm(x):
    return jnp.clip(jnp.sqrt(jnp.sum(x * x, -1, keepdims=True)), 1e-15)


def _proj(x):
    maxnorm = 1.0 - 1e-3
    n = _nrm(x)
    return jnp.where(n > maxnorm, x / n * maxnorm, x)


def _expmap0(u):
    un = _nrm(u)
    return jnp.tanh(un) * u / un


def _logmap0(p):
    pn = _nrm(p)
    return _artanh(pn) * p / pn


def _mobius_add(x, y):
    x2 = jnp.sum(x * x, -1, keepdims=True)
    y2 = jnp.sum(y * y, -1, keepdims=True)
    xy = jnp.sum(x * y, -1, keepdims=True)
    num = (1.0 + 2.0 * xy + y2) * x + (1.0 - x2) * y
    den = 1.0 + 2.0 * xy + x2 * y2
    return num / jnp.clip(den, 1e-15)


def _hyp_linear(h, wt, bvec):
    xn = _nrm(h)
    mx = jnp.dot(h, wt, preferred_element_type=jnp.float32)
    mxn = _nrm(mx)
    mv = _proj(jnp.tanh(mxn / xn * _artanh(xn)) * mx / mxn)
    hb = _proj(_expmap0(bvec[None, :]))
    res = _proj(_mobius_add(mv, hb))
    return _logmap0(res)


def _post_agg(agg_raw, w):
    agg = agg_raw * w[:, None]
    h2 = _proj(_expmap0(agg))
    xt2 = jax.nn.relu(_logmap0(h2))
    return _proj(_expmap0(xt2))


R = 1024
GRID = ACC_N // R


def _tc1_body(x_ref, wt_ref, b_ref, out_ref):
    h = _proj(_expmap0(x_ref[...]))
    xn = _nrm(h)
    mx_all = jnp.dot(h, wt_ref[...], preferred_element_type=jnp.float32)
    for s in range(2):
        mx = mx_all[:, s * D:(s + 1) * D]
        mxn = _nrm(mx)
        mv = _proj(jnp.tanh(mxn / xn * _artanh(xn)) * mx / mxn)
        hb = _proj(_expmap0(b_ref[s][None, :]))
        res = _proj(_mobius_add(mv, hb))
        out_ref[s] = _logmap0(res)


def _tc2_body(agg_ref, deg_ref, wt_ref, b_ref, out_ref):
    w = 1.0 / jnp.clip(deg_ref[...], 1.0)
    for s in range(2):
        h = _post_agg(agg_ref[s], w[s])
        out_ref[s] = _hyp_linear(h, wt_ref[s], b_ref[s])


def _tc3a_body(agga_ref, deg_ref, out_ref):
    w = 1.0 / jnp.clip(deg_ref[...], 1.0)
    for s in range(2):
        out_ref[s] = _post_agg(agga_ref[s], w[s])


def _tc3b_body(x1a_ref, aggb_ref, deg_ref, out_ref):
    w = 1.0 / jnp.clip(deg_ref[...], 1.0)
    x1 = ([x1a_ref[s] for s in range(2)]
          + [_post_agg(aggb_ref[s], w[s]) for s in range(2)])
    wgt = [_proj(_expmap0(0.125 * _logmap0(xi))) for xi in x1]
    target = _mobius_add(_mobius_add(_mobius_add(wgt[0], wgt[1]), wgt[2]), wgt[3])
    tt = [_logmap0(v) for v in (x1 + [target])]
    mean = (tt[0] + tt[1] + tt[2] + tt[3] + tt[4]) * 0.2
    out_ref[...] = _proj(_expmap0(mean))


def _tc1(x, wt_pair, b0_pair):
    return pl.pallas_call(
        _tc1_body,
        grid=(GRID,),
        in_specs=[
            pl.BlockSpec((R, D), lambda i: (i, 0)),
            pl.BlockSpec((D, 2 * D), lambda i: (0, 0)),
            pl.BlockSpec((2, D), lambda i: (0, 0)),
        ],
        out_specs=pl.BlockSpec((2, R, D), lambda i: (0, i, 0)),
        out_shape=jax.ShapeDtypeStruct((2, ACC_N, D), jnp.float32),
    )(x, wt_pair, b0_pair)


def _tc2(agg, deg, wt2, b1):
    return pl.pallas_call(
        _tc2_body,
        grid=(GRID,),
        in_specs=[
            pl.BlockSpec((2, R, D), lambda i: (0, i, 0)),
            pl.BlockSpec((2, R), lambda i: (0, i)),
            pl.BlockSpec((2, D, D), lambda i: (0, 0, 0)),
            pl.BlockSpec((2, D), lambda i: (0, 0)),
        ],
        out_specs=pl.BlockSpec((2, R, D), lambda i: (0, i, 0)),
        out_shape=jax.ShapeDtypeStruct((2, ACC_N, D), jnp.float32),
    )(agg, deg, wt2, b1)


def _tc3a(agga, dega):
    return pl.pallas_call(
        _tc3a_body,
        grid=(GRID,),
        in_specs=[
            pl.BlockSpec((2, R, D), lambda i: (0, i, 0)),
            pl.BlockSpec((2, R), lambda i: (0, i)),
        ],
        out_specs=pl.BlockSpec((2, R, D), lambda i: (0, i, 0)),
        out_shape=jax.ShapeDtypeStruct((2, ACC_N, D), jnp.float32),
    )(agga, dega)


def _tc3b(x1a, aggb, degb):
    return pl.pallas_call(
        _tc3b_body,
        grid=(GRID,),
        in_specs=[
            pl.BlockSpec((2, R, D), lambda i: (0, i, 0)),
            pl.BlockSpec((2, R, D), lambda i: (0, i, 0)),
            pl.BlockSpec((2, R), lambda i: (0, i)),
        ],
        out_specs=pl.BlockSpec((R, D), lambda i: (i, 0)),
        out_shape=jax.ShapeDtypeStruct((N, D), jnp.float32),
    )(x1a, aggb, degb)


GC = 40
NGRP = NCH // GC
GCD = 50


def _sc_agg_body(xt_hbm, src_hbm, dst_hbm, agg_hbm,
                 acc, sbuf, dbuf, r0, r1, g0, g1):
    cid = lax.axis_index("c")
    tid = lax.axis_index("s")

    for si in range(1):
        s = cid + si

        def _zero_rows(i, _):
            r0[i // 8, pl.ds((i % 8) * 16, 16)] = jnp.zeros((16,), jnp.float32)
            return 0
        lax.fori_loop(0, (K * D) // 16, _zero_rows, 0)
        for q in range(ZCH // K):
            pltpu.sync_copy(r0, acc.at[pl.ds(tid * ZCH + q * K, K), :])
        rem = ZCH - (ZCH // K) * K
        if rem:
            pltpu.sync_copy(r0.at[pl.ds(0, rem), :],
                            acc.at[pl.ds(tid * ZCH + (ZCH // K) * K, rem), :])
        plsc.subcore_barrier()

        for gl in range(NGRP):
            pltpu.sync_copy(src_hbm.at[s, tid, gl], sbuf)
            pltpu.sync_copy(dst_hbm.at[s, tid, gl], dbuf)
            pltpu.async_copy(xt_hbm.at[sbuf.at[0]], r0, g0)

            def _chunk_pair(w, _):
                a = 2 * w
                b = a + 1
                pltpu.make_async_copy(xt_hbm.at[sbuf.at[a]], r0, g0).wait()
                pltpu.async_copy(xt_hbm.at[sbuf.at[b]], r1, g1)
                pltpu.sync_copy(r0, acc.at[dbuf.at[a]], add=True)
                pltpu.make_async_copy(xt_hbm.at[sbuf.at[b]], r1, g1).wait()

                @pl.when(w < GC // 2 - 1)
                def _():
                    pltpu.async_copy(xt_hbm.at[sbuf.at[a + 2]], r0, g0)

                pltpu.sync_copy(r1, acc.at[dbuf.at[b]], add=True)
                return 0

            lax.fori_loop(0, GC // 2, _chunk_pair, 0)

        plsc.subcore_barrier()
        pltpu.sync_copy(acc.at[pl.ds(tid * ZCH, ZCH), :],
                        agg_hbm.at[s, pl.ds(tid * ZCH, ZCH), :])
        plsc.subcore_barrier()


def _sc_deg_body(dst_hbm, deg3_hbm, stage, ldeg, rbuf, obuf, dbuf):
    cid = lax.axis_index("c")
    tid = lax.axis_index("s")

    ones16 = jnp.ones((16,), jnp.float32)
    zeros16 = jnp.zeros((16,), jnp.float32)

    for si in range(2):
        s = cid * 2 + si

        def _zero(i, _):
            ldeg[pl.ds(i * 16, 16)] = zeros16
            return 0
        lax.fori_loop(0, ACC_N // 16, _zero, 0)

        for gl in range(NGRP):
            pltpu.sync_copy(dst_hbm.at[s, tid, gl], dbuf)

            def _count(f, _):
                iv = dbuf[f // 5, pl.ds((f % 5) * 16, 16)]
                plsc.addupdate_scatter(ldeg, [iv], ones16)
                return 0

            lax.fori_loop(0, (GCD * KD) // 16, _count, 0)

        pltpu.sync_copy(ldeg, stage.at[tid, 0])
        plsc.subcore_barrier()

        @pl.when(tid < ACC_N // 1024)
        def _():
            pltpu.sync_copy(stage.at[:, 0, pl.ds(tid * 1024, 1024)], rbuf)

            def _reduce(f, _):
                c = f * 16
                tot = rbuf[0, pl.ds(c, 16)]
                for t in range(1, NS):
                    tot = tot + rbuf[t, pl.ds(c, 16)]
                obuf[f // 8, pl.ds((f % 8) * 16, 16)] = tot
                return 0

            lax.fori_loop(0, 1024 // 16, _reduce, 0)
            pltpu.sync_copy(obuf, deg3_hbm.at[s, pl.ds(tid * 8, 8), :])

        plsc.subcore_barrier()


def _sc_mesh():
    return plsc.VectorSubcoreMesh(core_axis_name="c", subcore_axis_name="s",
                                  num_cores=NC, num_subcores=NS)


def _sc_agg(xt2d, srcoff, dstr):
    fn = pl.kernel(
        _sc_agg_body,
        out_type=(jax.ShapeDtypeStruct((2, ACC_N, D), jnp.float32),),
        mesh=_sc_mesh(),
        scratch_types=[
            pltpu.VMEM_SHARED((ACC_N, D), jnp.float32),
            pltpu.VMEM((GC, K), jnp.int32),
            pltpu.VMEM((GC, K), jnp.int32),
            pltpu.VMEM((K, D), jnp.float32),
            pltpu.VMEM((K, D), jnp.float32),
            pltpu.SemaphoreType.DMA,
            pltpu.SemaphoreType.DMA,
        ],
    )
    return fn(xt2d, srcoff, dstr)[0]


def _sc_deg(dstr):
    fn = pl.kernel(
        _sc_deg_body,
        out_type=(jax.ShapeDtypeStruct((NSETS, ACC_N // 128, 128), jnp.float32),),
        mesh=_sc_mesh(),
        compiler_params=pltpu.CompilerParams(needs_layout_passes=False),
        scratch_types=[
            pltpu.VMEM_SHARED((NS, 1, ACC_N), jnp.float32),
            pltpu.VMEM((ACC_N,), jnp.float32),
            pltpu.VMEM((NS, 1024), jnp.float32),
            pltpu.VMEM((8, 128), jnp.float32),
            pltpu.VMEM((GCD, KD), jnp.int32),
        ],
    )
    return fn(dstr)[0]


def kernel(x, W, b, ei_d_i, ei_d_o, ei_n_i, ei_n_o):
    ei = jnp.stack([ei_d_i, ei_d_o, ei_n_i, ei_n_o])
    srcoff = (ei[:, 0, :]
              + ((jnp.arange(NSETS, dtype=jnp.int32) % 2) * ACC_N)[:, None])
    srcoff = srcoff.reshape(NSETS, NS, NGRP, GC, K)
    dstr = ei[:, 1, :].reshape(NSETS, NS, NGRP, GC, K)
    dstr_deg = ei[:, 1, :].reshape(NSETS, NS, NGRP, GCD, KD)

    wt1a = jnp.concatenate([W[0, 0].T, W[1, 0].T], axis=1)
    wt1b = jnp.concatenate([W[2, 0].T, W[3, 0].T], axis=1)
    wt2 = W[:, 1].transpose(0, 2, 1)

    deg = _sc_deg(dstr_deg).reshape(NSETS, ACC_N)
    xt0a = _tc1(x, wt1a, b[0:2, 0])
    agg1a = _sc_agg(xt0a.reshape(2 * ACC_N, D), srcoff[0:2], dstr[0:2])
    xt0b = _tc1(x, wt1b, b[2:4, 0])
    agg1b = _sc_agg(xt0b.reshape(2 * ACC_N, D), srcoff[2:4], dstr[2:4])
    xt1a = _tc2(agg1a, deg[0:2], wt2[0:2], b[0:2, 1])
    agg2a = _sc_agg(xt1a.reshape(2 * ACC_N, D), srcoff[0:2], dstr[0:2])
    xt1b = _tc2(agg1b, deg[2:4], wt2[2:4], b[2:4, 1])
    agg2b = _sc_agg(xt1b.reshape(2 * ACC_N, D), srcoff[2:4], dstr[2:4])
    x1a = _tc3a(agg2a, deg[0:2])
    return _tc3b(x1a, agg2b, deg[2:4])


# speedup vs baseline: 1.9030x; 1.0077x over previous
import functools

import jax
import jax.numpy as jnp
from jax import lax
from jax.experimental import pallas as pl
from jax.experimental.pallas import tpu as pltpu
from jax.experimental.pallas import tpu_sc as plsc

N = 10000
D = 128
E = 320000
NSETS = 4

NC = 2
NS = 16
K = 100
KD = 80
NCH = E // (NS * K)
ACC_N = 10240
ZCH = ACC_N // NS


def _artanh(x):
    x = jnp.clip(x, -1.0 + 1e-7, 1.0 - 1e-7)
    return 0.5 * jnp.log((1.0 + x) / (1.0 - x))


def _nrm(x):
    return jnp.clip(jnp.sqrt(jnp.sum(x * x, -1, keepdims=True)), 1e-15)


def _proj(x):
    maxnorm = 1.0 - 1e-3
    n = _nrm(x)
    return jnp.where(n > maxnorm, x / n * maxnorm, x)


def _expmap0(u):
    un = _nrm(u)
    return jnp.tanh(un) * u / un


def _logmap0(p):
    pn = _nrm(p)
    return _artanh(pn) * p / pn


def _mobius_add(x, y):
    x2 = jnp.sum(x * x, -1, keepdims=True)
    y2 = jnp.sum(y * y, -1, keepdims=True)
    xy = jnp.sum(x * y, -1, keepdims=True)
    num = (1.0 + 2.0 * xy + y2) * x + (1.0 - x2) * y
    den = 1.0 + 2.0 * xy + x2 * y2
    return num / jnp.clip(den, 1e-15)


def _hyp_linear(h, wt, bvec):
    xn = _nrm(h)
    mx = jnp.dot(h, wt, preferred_element_type=jnp.float32)
    mxn = _nrm(mx)
    mv = _proj(jnp.tanh(mxn / xn * _artanh(xn)) * mx / mxn)
    hb = _proj(_expmap0(bvec[None, :]))
    res = _proj(_mobius_add(mv, hb))
    return _logmap0(res)


def _post_agg(agg_raw, w):
    agg = agg_raw * w[:, None]
    h2 = _proj(_expmap0(agg))
    xt2 = jax.nn.relu(_logmap0(h2))
    return _proj(_expmap0(xt2))


R = 2048
GRID = ACC_N // R


def _tc1_body(x_ref, wt_ref, b_ref, out_ref):
    h = _proj(_expmap0(x_ref[...]))
    xn = _nrm(h)
    mx_all = jnp.dot(h, wt_ref[...], preferred_element_type=jnp.float32)
    for s in range(2):
        mx = mx_all[:, s * D:(s + 1) * D]
        mxn = _nrm(mx)
        mv = _proj(jnp.tanh(mxn / xn * _artanh(xn)) * mx / mxn)
        hb = _proj(_expmap0(b_ref[s][None, :]))
        res = _proj(_mobius_add(mv, hb))
        out_ref[s] = _logmap0(res)


def _tc2_body(agg_ref, deg_ref, wt_ref, b_ref, out_ref):
    w = 1.0 / jnp.clip(deg_ref[...], 1.0)
    for s in range(2):
        h = _post_agg(agg_ref[s], w[s])
        out_ref[s] = _hyp_linear(h, wt_ref[s], b_ref[s])


def _tc3a_body(agga_ref, deg_ref, out_ref):
    w = 1.0 / jnp.clip(deg_ref[...], 1.0)
    for s in range(2):
        out_ref[s] = _post_agg(agga_ref[s], w[s])


def _tc3b_body(x1a_ref, aggb_ref, deg_ref, out_ref):
    w = 1.0 / jnp.clip(deg_ref[...], 1.0)
    x1 = ([x1a_ref[s] for s in range(2)]
          + [_post_agg(aggb_ref[s], w[s]) for s in range(2)])
    wgt = [_proj(_expmap0(0.125 * _logmap0(xi))) for xi in x1]
    target = _mobius_add(_mobius_add(_mobius_add(wgt[0], wgt[1]), wgt[2]), wgt[3])
    tt = [_logmap0(v) for v in (x1 + [target])]
    mean = (tt[0] + tt[1] + tt[2] + tt[3] + tt[4]) * 0.2
    out_ref[...] = _proj(_expmap0(mean))


def _tc1(x, wt_pair, b0_pair):
    return pl.pallas_call(
        _tc1_body,
        grid=(GRID,),
        in_specs=[
            pl.BlockSpec((R, D), lambda i: (i, 0)),
            pl.BlockSpec((D, 2 * D), lambda i: (0, 0)),
            pl.BlockSpec((2, D), lambda i: (0, 0)),
        ],
        out_specs=pl.BlockSpec((2, R, D), lambda i: (0, i, 0)),
        out_shape=jax.ShapeDtypeStruct((2, ACC_N, D), jnp.float32),
    )(x, wt_pair, b0_pair)


def _tc2(agg, deg, wt2, b1):
    return pl.pallas_call(
        _tc2_body,
        grid=(GRID,),
        in_specs=[
            pl.BlockSpec((2, R, D), lambda i: (0, i, 0)),
            pl.BlockSpec((2, R), lambda i: (0, i)),
            pl.BlockSpec((2, D, D), lambda i: (0, 0, 0)),
            pl.BlockSpec((2, D), lambda i: (0, 0)),
        ],
        out_specs=pl.BlockSpec((2, R, D), lambda i: (0, i, 0)),
        out_shape=jax.ShapeDtypeStruct((2, ACC_N, D), jnp.float32),
    )(agg, deg, wt2, b1)


def _tc3a(agga, dega):
    return pl.pallas_call(
        _tc3a_body,
        grid=(GRID,),
        in_specs=[
            pl.BlockSpec((2, R, D), lambda i: (0, i, 0)),
            pl.BlockSpec((2, R), lambda i: (0, i)),
        ],
        out_specs=pl.BlockSpec((2, R, D), lambda i: (0, i, 0)),
        out_shape=jax.ShapeDtypeStruct((2, ACC_N, D), jnp.float32),
    )(agga, dega)


def _tc3b(x1a, aggb, degb):
    return pl.pallas_call(
        _tc3b_body,
        grid=(GRID,),
        in_specs=[
            pl.BlockSpec((2, R, D), lambda i: (0, i, 0)),
            pl.BlockSpec((2, R, D), lambda i: (0, i, 0)),
            pl.BlockSpec((2, R), lambda i: (0, i)),
        ],
        out_specs=pl.BlockSpec((R, D), lambda i: (i, 0)),
        out_shape=jax.ShapeDtypeStruct((N, D), jnp.float32),
    )(x1a, aggb, degb)


GC = 50
NGRP = NCH // GC
GCD = 50
NGRPD = E // (NS * GCD * KD)


def _sc_agg_body(xt_hbm, src_hbm, dst_hbm, agg_hbm,
                 acc, sbuf, dbuf, r0, r1, g0, g1):
    cid = lax.axis_index("c")
    tid = lax.axis_index("s")

    for si in range(1):
        s = cid + si

        def _zero_rows(i, _):
            r0[i // 8, pl.ds((i % 8) * 16, 16)] = jnp.zeros((16,), jnp.float32)
            return 0
        lax.fori_loop(0, (K * D) // 16, _zero_rows, 0)
        for q in range(ZCH // K):
            pltpu.sync_copy(r0, acc.at[pl.ds(tid * ZCH + q * K, K), :])
        rem = ZCH - (ZCH // K) * K
        if rem:
            pltpu.sync_copy(r0.at[pl.ds(0, rem), :],
                            acc.at[pl.ds(tid * ZCH + (ZCH // K) * K, rem), :])
        plsc.subcore_barrier()

        for gl in range(NGRP):
            pltpu.sync_copy(src_hbm.at[s, tid, gl], sbuf)
            pltpu.sync_copy(dst_hbm.at[s, tid, gl], dbuf)
            pltpu.async_copy(xt_hbm.at[sbuf.at[0]], r0, g0)

            def _chunk_pair(w, _):
                a = 2 * w
                b = a + 1
                pltpu.make_async_copy(xt_hbm.at[sbuf.at[a]], r0, g0).wait()
                pltpu.async_copy(xt_hbm.at[sbuf.at[b]], r1, g1)
                pltpu.sync_copy(r0, acc.at[dbuf.at[a]], add=True)
                pltpu.make_async_copy(xt_hbm.at[sbuf.at[b]], r1, g1).wait()

                @pl.when(w < GC // 2 - 1)
                def _():
                    pltpu.async_copy(xt_hbm.at[sbuf.at[a + 2]], r0, g0)

                pltpu.sync_copy(r1, acc.at[dbuf.at[b]], add=True)
                return 0

            lax.fori_loop(0, GC // 2, _chunk_pair, 0)

        plsc.subcore_barrier()
        pltpu.sync_copy(acc.at[pl.ds(tid * ZCH, ZCH), :],
                        agg_hbm.at[s, pl.ds(tid * ZCH, ZCH), :])
        plsc.subcore_barrier()


def _sc_deg_body(dst_hbm, deg3_hbm, stage, ldeg, rbuf, obuf, dbuf):
    cid = lax.axis_index("c")
    tid = lax.axis_index("s")

    ones16 = jnp.ones((16,), jnp.float32)
    zeros16 = jnp.zeros((16,), jnp.float32)

    for si in range(2):
        s = cid * 2 + si

        def _zero(i, _):
            ldeg[pl.ds(i * 16, 16)] = zeros16
            return 0
        lax.fori_loop(0, ACC_N // 16, _zero, 0)

        for gl in range(NGRPD):
            pltpu.sync_copy(dst_hbm.at[s, tid, gl], dbuf)

            def _count(f, _):
                iv = dbuf[f // 5, pl.ds((f % 5) * 16, 16)]
                plsc.addupdate_scatter(ldeg, [iv], ones16)
                return 0

            lax.fori_loop(0, (GCD * KD) // 16, _count, 0)

        pltpu.sync_copy(ldeg, stage.at[tid, 0])
        plsc.subcore_barrier()

        @pl.when(tid < ACC_N // 1024)
        def _():
            pltpu.sync_copy(stage.at[:, 0, pl.ds(tid * 1024, 1024)], rbuf)

            def _reduce(f, _):
                c = f * 16
                tot = rbuf[0, pl.ds(c, 16)]
                for t in range(1, NS):
                    tot = tot + rbuf[t, pl.ds(c, 16)]
                obuf[f // 8, pl.ds((f % 8) * 16, 16)] = tot
                return 0

            lax.fori_loop(0, 1024 // 16, _reduce, 0)
            pltpu.sync_copy(obuf, deg3_hbm.at[s, pl.ds(tid * 8, 8), :])

        plsc.subcore_barrier()


def _sc_mesh():
    return plsc.VectorSubcoreMesh(core_axis_name="c", subcore_axis_name="s",
                                  num_cores=NC, num_subcores=NS)


def _sc_agg(xt2d, srcoff, dstr):
    fn = pl.kernel(
        _sc_agg_body,
        out_type=(jax.ShapeDtypeStruct((2, ACC_N, D), jnp.float32),),
        mesh=_sc_mesh(),
        scratch_types=[
            pltpu.VMEM_SHARED((ACC_N, D), jnp.float32),
            pltpu.VMEM((GC, K), jnp.int32),
            pltpu.VMEM((GC, K), jnp.int32),
            pltpu.VMEM((K, D), jnp.float32),
            pltpu.VMEM((K, D), jnp.float32),
            pltpu.SemaphoreType.DMA,
            pltpu.SemaphoreType.DMA,
        ],
    )
    return fn(xt2d, srcoff, dstr)[0]


def _sc_deg(dstr):
    fn = pl.kernel(
        _sc_deg_body,
        out_type=(jax.ShapeDtypeStruct((NSETS, ACC_N // 128, 128), jnp.float32),),
        mesh=_sc_mesh(),
        compiler_params=pltpu.CompilerParams(needs_layout_passes=False),
        scratch_types=[
            pltpu.VMEM_SHARED((NS, 1, ACC_N), jnp.float32),
            pltpu.VMEM((ACC_N,), jnp.float32),
            pltpu.VMEM((NS, 1024), jnp.float32),
            pltpu.VMEM((8, 128), jnp.float32),
            pltpu.VMEM((GCD, KD), jnp.int32),
        ],
    )
    return fn(dstr)[0]


def kernel(x, W, b, ei_d_i, ei_d_o, ei_n_i, ei_n_o):
    ei = jnp.stack([ei_d_i, ei_d_o, ei_n_i, ei_n_o])
    srcoff = (ei[:, 0, :]
              + ((jnp.arange(NSETS, dtype=jnp.int32) % 2) * ACC_N)[:, None])
    srcoff = srcoff.reshape(NSETS, NS, NGRP, GC, K)
    dstr = ei[:, 1, :].reshape(NSETS, NS, NGRP, GC, K)
    dstr_deg = ei[:, 1, :].reshape(NSETS, NS, NGRPD, GCD, KD)

    wt1a = jnp.concatenate([W[0, 0].T, W[1, 0].T], axis=1)
    wt1b = jnp.concatenate([W[2, 0].T, W[3, 0].T], axis=1)
    wt2 = W[:, 1].transpose(0, 2, 1)

    deg = _sc_deg(dstr_deg).reshape(NSETS, ACC_N)
    xt0a = _tc1(x, wt1a, b[0:2, 0])
    agg1a = _sc_agg(xt0a.reshape(2 * ACC_N, D), srcoff[0:2], dstr[0:2])
    xt0b = _tc1(x, wt1b, b[2:4, 0])
    agg1b = _sc_agg(xt0b.reshape(2 * ACC_N, D), srcoff[2:4], dstr[2:4])
    xt1a = _tc2(agg1a, deg[0:2], wt2[0:2], b[0:2, 1])
    agg2a = _sc_agg(xt1a.reshape(2 * ACC_N, D), srcoff[0:2], dstr[0:2])
    xt1b = _tc2(agg1b, deg[2:4], wt2[2:4], b[2:4, 1])
    agg2b = _sc_agg(xt1b.reshape(2 * ACC_N, D), srcoff[2:4], dstr[2:4])
    x1a = _tc3a(agg2a, deg[0:2])
    return _tc3b(x1a, agg2b, deg[2:4])
